# baseline (speedup 1.0000x reference)
import numpy as np
import jax
import jax.numpy as jnp
from jax import lax
from jax.experimental import pallas as pl
from jax.experimental.pallas import tpu as pltpu
from jax.experimental.pallas import tpu_sc as plsc

HID = 256
NHEADS = 8
HD = 32
NNODES = 10000
NEDGES = 160000

F32 = jnp.float32


def _pack_bf16_pairs(x):
    lo = lax.bitcast_convert_type(x[:, :128], jnp.uint32)
    hi = lax.bitcast_convert_type(x[:, 128:], jnp.uint32)
    lo16 = (lo + 0x8000) >> 16
    hi16 = (hi + 0x8000) & jnp.uint32(0xFFFF0000)
    return lax.bitcast_convert_type(lo16 | hi16, F32)


def _unpack_bf16_pairs(p):
    u = lax.bitcast_convert_type(p, jnp.uint32)
    lo = lax.bitcast_convert_type(u << 16, F32)
    hi = lax.bitcast_convert_type(u & jnp.uint32(0xFFFF0000), F32)
    return jnp.concatenate([lo, hi], axis=1)


def _stage_a_body(nodes, nenc, wf, ws, b12, wno, weo, beo, bno,
                  n2_o, nout_o, wc_o, c2_o):
    n = nodes[...] + nenc[...]
    w12 = wf[...] + ws[...]
    n2 = jnp.dot(n, w12, preferred_element_type=F32) + b12[...]
    n2_o[...] = _pack_bf16_pairs(n2)
    n_out = jnp.dot(n, wno[...], preferred_element_type=F32)
    nout_o[...] = _pack_bf16_pairs(n_out)
    wc_o[...] = jnp.dot(weo[...], wno[...], preferred_element_type=F32)
    c2_o[...] = jnp.dot(beo[...], wno[...], preferred_element_type=F32) + bno[...]


def _stage_a(nodes, nenc, wf, ws, b12, wno, weo, beo, bno):
    return pl.pallas_call(
        _stage_a_body,
        out_shape=(
            jax.ShapeDtypeStruct((NNODES, 128), F32),
            jax.ShapeDtypeStruct((NNODES, 128), F32),
            jax.ShapeDtypeStruct((HID, HID), F32),
            jax.ShapeDtypeStruct((1, HID), F32),
        ),
    )(nodes, nenc, wf, ws, b12, wno, weo, beo, bno)


GC = 128
NCHUNK = NEDGES // GC
NW = 32


def _gather_body(n2_hbm, nout_hbm, seg_hbm, idx1_hbm, g1_hbm, g2_hbm,
                 idx1_v, idx2_v, rows1_v, rows2_v,
                 sem_g1, sem_g2, sem_w1, sem_w2):
    wid = lax.axis_index("s") * 2 + lax.axis_index("c")

    def body(i, carry):
        chunk = wid + NW * i

        @pl.when(chunk < NCHUNK)
        def _():
            base = chunk * GC
            pltpu.sync_copy(seg_hbm.at[pl.ds(base, GC)], idx1_v)
            g1 = pltpu.async_copy(n2_hbm.at[idx1_v], rows1_v, sem_g1)
            pltpu.sync_copy(idx1_hbm.at[pl.ds(base, GC)], idx2_v)
            g2 = pltpu.async_copy(nout_hbm.at[idx2_v], rows2_v, sem_g2)
            g1.wait()
            w1 = pltpu.async_copy(rows1_v, g1_hbm.at[pl.ds(base, GC)], sem_w1)
            g2.wait()
            w2 = pltpu.async_copy(rows2_v, g2_hbm.at[pl.ds(base, GC)], sem_w2)
            w1.wait()
            w2.wait()

        return carry

    lax.fori_loop(0, (NCHUNK + NW - 1) // NW, body, 0)


def _sc_gather(n2, n_out, seg, idx1):
    mesh = plsc.VectorSubcoreMesh(core_axis_name="c", subcore_axis_name="s")
    return pl.kernel(
        _gather_body,
        out_type=(
            jax.ShapeDtypeStruct((NEDGES, 128), F32),
            jax.ShapeDtypeStruct((NEDGES, 128), F32),
        ),
        mesh=mesh,
        scratch_types=[
            pltpu.VMEM((GC,), jnp.int32),
            pltpu.VMEM((GC,), jnp.int32),
            pltpu.VMEM((GC, 128), F32),
            pltpu.VMEM((GC, 128), F32),
            pltpu.SemaphoreType.DMA,
            pltpu.SemaphoreType.DMA,
            pltpu.SemaphoreType.DMA,
            pltpu.SemaphoreType.DMA,
        ],
    )(n2, n_out, seg, idx1)


EBLK = 4000
_INV_SQRT2 = 0.7071067811865476


def _stage_c_body(edges, eenc, g1, g2, we, be, p_mat, weo, beo, wc, c2, r_mat,
                  ne_o, scaled_o, w_o):
    e = edges[...] + eenc[...]
    ep = jnp.dot(e, we[...], preferred_element_type=F32) + be[...]
    x = _unpack_bf16_pairs(g1[...]) + ep
    h = x * 0.5 * (1.0 + lax.erf(x * _INV_SQRT2))
    logits = jnp.dot(h, p_mat[...], preferred_element_type=F32)
    w = jnp.exp(logits)
    ne_o[...] = jnp.dot(h, weo[...], preferred_element_type=F32) + beo[...]
    vpre = (_unpack_bf16_pairs(g2[...])
            + jnp.dot(h, wc[...], preferred_element_type=F32) + c2[...])
    wrep = jnp.dot(w, r_mat[...], preferred_element_type=F32)
    scaled = vpre * wrep
    scaled_o[0, :, :] = scaled[:, :128]
    scaled_o[1, :, :] = scaled[:, 128:]
    w_o[...] = w


def _stage_c(edges, eenc, g1, g2, we, be, p_mat, weo, beo, wc, c2, r_mat):
    grid = (NEDGES // EBLK,)
    eb = lambda i: (i, 0)
    full = lambda i: (0, 0)
    return pl.pallas_call(
        _stage_c_body,
        grid=grid,
        in_specs=[
            pl.BlockSpec((EBLK, HID), eb),
            pl.BlockSpec((EBLK, HID), eb),
            pl.BlockSpec((EBLK, 128), eb),
            pl.BlockSpec((EBLK, 128), eb),
            pl.BlockSpec((HID, HID), full),
            pl.BlockSpec((1, HID), full),
            pl.BlockSpec((HID, 16), full),
            pl.BlockSpec((HID, HID), full),
            pl.BlockSpec((1, HID), full),
            pl.BlockSpec((HID, HID), full),
            pl.BlockSpec((1, HID), full),
            pl.BlockSpec((16, HID), full),
        ],
        out_specs=[
            pl.BlockSpec((EBLK, HID), eb),
            pl.BlockSpec((2, EBLK, 128), lambda i: (0, i, 0)),
            pl.BlockSpec((EBLK, 16), eb),
        ],
        out_shape=(
            jax.ShapeDtypeStruct((NEDGES, HID), F32),
            jax.ShapeDtypeStruct((2, NEDGES, 128), F32),
            jax.ShapeDtypeStruct((NEDGES, 16), F32),
        ),
    )(edges, eenc, g1, g2, we, be, p_mat, weo, beo, wc, c2, r_mat)


SGC = 64
SNCHUNK = NEDGES // SGC
ENODES = NNODES // 8
SZCHUNK = NNODES // SGC
SZTAIL = NNODES - SZCHUNK * SGC
SNHJ = SGC // 16


def _scatter_body(scaled_hbm, wflat_hbm, seg_hbm, z128_hbm, zden_hbm,
                  num_o, den_o,
                  idx0_v, idx1_v, rows0_v, rows1_v, wf0_v, wf1_v, den_acc,
                  num_sp,
                  sem_i0, sem_i1, sem_w0, sem_w1, sem_r0, sem_r1,
                  sem_s0, sem_s1):
    c = lax.axis_index("c")
    s = lax.axis_index("s")
    slots = ((idx0_v, rows0_v, wf0_v, sem_i0, sem_w0, sem_r0, sem_s0),
             (idx1_v, rows1_v, wf1_v, sem_i1, sem_w1, sem_r1, sem_s1))

    pltpu.sync_copy(z128_hbm.at[pl.ds(0, SGC)], rows0_v)

    def zbody(i, carry):
        chunk = s + 16 * i

        @pl.when(chunk < SZCHUNK)
        def _():
            pltpu.sync_copy(rows0_v, num_sp.at[pl.ds(chunk * SGC, SGC)])

        return carry

    lax.fori_loop(0, (SZCHUNK + 15) // 16, zbody, 0)

    @pl.when(s == 15)
    def _():
        pltpu.sync_copy(rows0_v.at[pl.ds(0, SZTAIL)],
                        num_sp.at[pl.ds(SZCHUNK * SGC, SZTAIL)])

    plsc.subcore_barrier()

    lane = lax.iota(jnp.int32, 16)

    def chunk_of(k):
        return s + 16 * k

    def start_loads(k, slot, with_num):
        idx_v, rows_v, wf_v, sem_i, sem_w, sem_r, _ = slots[slot]

        @pl.when(chunk_of(k) < SNCHUNK)
        def _():
            base = chunk_of(k) * SGC
            pltpu.async_copy(seg_hbm.at[pl.ds(base, SGC)], idx_v, sem_i)
            pltpu.async_copy(wflat_hbm.at[pl.ds(base * 16, SGC * 16)],
                             wf_v, sem_w)
            if with_num:
                pltpu.async_copy(scaled_hbm.at[c, pl.ds(base, SGC)],
                                 rows_v, sem_r)

    def run_pass(q, with_num):
        region_base = (4 * c + q) * ENODES
        pltpu.sync_copy(zden_hbm, den_acc)
        for slot in (0, 1):
            start_loads(slot, slot, with_num)

        def half_step(k, slot):
            idx_v, rows_v, wf_v, sem_i, sem_w, sem_r, sem_s = slots[slot]

            @pl.when(chunk_of(k) < SNCHUNK)
            def _():
                base = chunk_of(k) * SGC
                pltpu.make_async_copy(seg_hbm.at[pl.ds(base, SGC)], idx_v,
                                      sem_i).wait()
                pltpu.make_async_copy(
                    wflat_hbm.at[pl.ds(base * 16, SGC * 16)], wf_v,
                    sem_w).wait()
                if with_num:
                    pltpu.make_async_copy(scaled_hbm.at[c, pl.ds(base, SGC)],
                                          rows_v, sem_r).wait()
                    pltpu.async_copy(rows_v, num_sp.at[idx_v], sem_s,
                                     add=True)

                def dbody(j, carry2):
                    ids = idx_v[pl.ds(j * 16, 16)]
                    t = ids - region_base
                    ok = (t >= 0) & (t < ENODES)
                    tc16 = jnp.where(ok, t, 0) * 16
                    rows16 = (j * 16 + lane) * 16
                    for h in range(NHEADS):
                        vals = plsc.load_gather(wf_v, [rows16 + h])
                        plsc.addupdate_scatter(den_acc, [tc16 + h], vals,
                                               mask=ok)
                    return carry2

                lax.fori_loop(0, SNHJ, dbody, 0)
                if with_num:
                    pltpu.make_async_copy(rows_v, num_sp.at[idx_v],
                                          sem_s).wait()
                start_loads(k + 2, slot, with_num)

        def body(m, carry):
            half_step(2 * m, 0)
            half_step(2 * m + 1, 1)
            return carry

        nk = (SNCHUNK + 15) // 16
        lax.fori_loop(0, (nk + 1) // 2, body, 0)
        pltpu.sync_copy(den_acc, den_o.at[c, q, s])

    run_pass(0, True)
    for q in (1, 2, 3):
        run_pass(q, False)

    plsc.subcore_barrier()

    def fbody(i, carry):
        chunk = s + 16 * i

        @pl.when(chunk < SZCHUNK)
        def _():
            base = chunk * SGC
            pltpu.sync_copy(num_sp.at[pl.ds(base, SGC)], rows0_v)
            pltpu.sync_copy(rows0_v, num_o.at[c, pl.ds(base, SGC)])

        return carry

    lax.fori_loop(0, (SZCHUNK + 15) // 16, fbody, 0)

    @pl.when(s == 15)
    def _():
        pltpu.sync_copy(num_sp.at[pl.ds(SZCHUNK * SGC, SZTAIL)],
                        rows0_v.at[pl.ds(0, SZTAIL)])
        pltpu.sync_copy(rows0_v.at[pl.ds(0, SZTAIL)],
                        num_o.at[c, pl.ds(SZCHUNK * SGC, SZTAIL)])


def _sc_scatter(scaled2, wflat, seg, z128, zden):
    mesh = plsc.VectorSubcoreMesh(core_axis_name="c", subcore_axis_name="s")
    return pl.kernel(
        _scatter_body,
        out_type=(
            jax.ShapeDtypeStruct((2, NNODES, 128), F32),
            jax.ShapeDtypeStruct((2, 4, 16, ENODES * 16), F32),
        ),
        mesh=mesh,
        compiler_params=pltpu.CompilerParams(needs_layout_passes=False),
        scratch_types=[
            pltpu.VMEM((SGC,), jnp.int32),
            pltpu.VMEM((SGC,), jnp.int32),
            pltpu.VMEM((SGC, 128), F32),
            pltpu.VMEM((SGC, 128), F32),
            pltpu.VMEM((SGC * 16,), F32),
            pltpu.VMEM((SGC * 16,), F32),
            pltpu.VMEM((ENODES * 16,), F32),
            pltpu.VMEM_SHARED((NNODES, 128), F32),
            pltpu.SemaphoreType.DMA,
            pltpu.SemaphoreType.DMA,
            pltpu.SemaphoreType.DMA,
            pltpu.SemaphoreType.DMA,
            pltpu.SemaphoreType.DMA,
            pltpu.SemaphoreType.DMA,
            pltpu.SemaphoreType.DMA,
            pltpu.SemaphoreType.DMA,
        ],
    )(scaled2, wflat, seg, z128, zden)


def _stage_d2_body(parts, den_o):
    den_o[0] = jnp.sum(parts[0, 0], axis=0)


def _stage_d2(den_parts):
    return pl.pallas_call(
        _stage_d2_body,
        grid=(8,),
        in_specs=[pl.BlockSpec((1, 1, 16, ENODES, 16),
                               lambda i: (i // 4, i % 4, 0, 0, 0))],
        out_specs=pl.BlockSpec((1, ENODES, 16), lambda i: (i, 0, 0)),
        out_shape=jax.ShapeDtypeStruct((8, ENODES, 16), F32),
    )(den_parts)


NBLK = 2000


def _stage_e_body(num0, num1, den, r0, r1, out_o):
    d = den[...]
    d0 = jnp.dot(d, r0[...], preferred_element_type=F32)
    d1 = jnp.dot(d, r1[...], preferred_element_type=F32)
    rec0 = jnp.where(d0 > 0, 1.0 / d0, 0.0)
    rec1 = jnp.where(d1 > 0, 1.0 / d1, 0.0)
    out_o[:, 0:128] = num0[0] * rec0
    out_o[:, 128:256] = num1[0] * rec1


def _stage_e(num2, den, r0, r1):
    grid = (NNODES // NBLK,)
    return pl.pallas_call(
        _stage_e_body,
        grid=grid,
        in_specs=[
            pl.BlockSpec((1, NBLK, 128), lambda i: (0, i, 0)),
            pl.BlockSpec((1, NBLK, 128), lambda i: (1, i, 0)),
            pl.BlockSpec((NBLK, 16), lambda i: (i, 0)),
            pl.BlockSpec((16, 128), lambda i: (0, 0)),
            pl.BlockSpec((16, 128), lambda i: (0, 0)),
        ],
        out_specs=pl.BlockSpec((NBLK, HID), lambda i: (i, 0)),
        out_shape=jax.ShapeDtypeStruct((NNODES, HID), F32),
    )(num2, num2, den, r0, r1)


_R_MAT_NP = np.zeros((16, HID), np.float32)
for _h in range(NHEADS):
    _R_MAT_NP[_h, _h * HD:(_h + 1) * HD] = 1.0
_R0_NP = np.zeros((16, 128), np.float32)
for _h in range(4):
    _R0_NP[_h, _h * HD:(_h + 1) * HD] = 1.0
_R1_NP = np.zeros((16, 128), np.float32)
for _h in range(4):
    _R1_NP[_h + 4, _h * HD:(_h + 1) * HD] = 1.0


def kernel(nodes, edges, edge_index, node_encodings, edge_encodings,
           W_first, b_first, W_second, b_second, W_edge, b_edge,
           attn_proj, W_edge_out, b_edge_out, W_node_out, b_node_out):
    seg = edge_index[0]
    idx1 = edge_index[1]

    b12 = (b_first + b_second).reshape(1, HID)
    be = b_edge.reshape(1, HID)
    beo = b_edge_out.reshape(1, HID)
    bno = b_node_out.reshape(1, HID)

    a = attn_proj[0]
    p_mat = (a[:, :, None] * jnp.eye(NHEADS, dtype=F32)[:, None, :])
    p_mat = p_mat.reshape(HID, NHEADS)
    p_mat = jnp.concatenate([p_mat, jnp.zeros((HID, 8), F32)], axis=1)

    n2, n_out, wc, c2 = _stage_a(nodes, node_encodings, W_first, W_second,
                                 b12, W_node_out, W_edge_out, beo, bno)
    g1, g2 = _sc_gather(n2, n_out, seg, idx1)
    new_edges, scaled2, w16 = _stage_c(edges, edge_encodings, g1, g2,
                                       W_edge, be, p_mat, W_edge_out, beo,
                                       wc, c2, jnp.asarray(_R_MAT_NP))
    z128 = jnp.zeros((NNODES, 128), F32)
    zden = jnp.zeros((ENODES * 16,), F32)
    num2, den_parts = _sc_scatter(scaled2, w16.reshape(-1), seg, z128, zden)
    den = _stage_d2(
        den_parts.reshape(2, 4, 16, ENODES, 16)).reshape(NNODES, 16)
    new_nodes = _stage_e(num2, den, jnp.asarray(_R0_NP), jnp.asarray(_R1_NP))
    return new_nodes, new_edges


# speedup vs baseline: 1.0189x; 1.0189x over previous
import numpy as np
import jax
import jax.numpy as jnp
from jax import lax
from jax.experimental import pallas as pl
from jax.experimental.pallas import tpu as pltpu
from jax.experimental.pallas import tpu_sc as plsc

HID = 256
NHEADS = 8
HD = 32
NNODES = 10000
NEDGES = 160000

F32 = jnp.float32


def _pack_bf16_pairs(x):
    lo = lax.bitcast_convert_type(x[:, :128], jnp.uint32)
    hi = lax.bitcast_convert_type(x[:, 128:], jnp.uint32)
    lo16 = (lo + 0x8000) >> 16
    hi16 = (hi + 0x8000) & jnp.uint32(0xFFFF0000)
    return lax.bitcast_convert_type(lo16 | hi16, F32)


def _unpack_bf16_pairs(p):
    u = lax.bitcast_convert_type(p, jnp.uint32)
    lo = lax.bitcast_convert_type(u << 16, F32)
    hi = lax.bitcast_convert_type(u & jnp.uint32(0xFFFF0000), F32)
    return jnp.concatenate([lo, hi], axis=1)


def _stage_a_body(nodes, nenc, wf, ws, b12, wno, weo, beo, bno,
                  n2_o, nout_o, wc_o, c2_o):
    n = nodes[...] + nenc[...]
    w12 = wf[...] + ws[...]
    n2 = jnp.dot(n, w12, preferred_element_type=F32) + b12[...]
    n2_o[...] = _pack_bf16_pairs(n2)
    n_out = jnp.dot(n, wno[...], preferred_element_type=F32)
    nout_o[...] = _pack_bf16_pairs(n_out)
    wc_o[...] = jnp.dot(weo[...], wno[...], preferred_element_type=F32)
    c2_o[...] = jnp.dot(beo[...], wno[...], preferred_element_type=F32) + bno[...]


def _stage_a(nodes, nenc, wf, ws, b12, wno, weo, beo, bno):
    return pl.pallas_call(
        _stage_a_body,
        out_shape=(
            jax.ShapeDtypeStruct((NNODES, 128), F32),
            jax.ShapeDtypeStruct((NNODES, 128), F32),
            jax.ShapeDtypeStruct((HID, HID), F32),
            jax.ShapeDtypeStruct((1, HID), F32),
        ),
    )(nodes, nenc, wf, ws, b12, wno, weo, beo, bno)


GC = 256
NCHUNK = NEDGES // GC
NW = 32


def _gather_body(n2_hbm, nout_hbm, seg_hbm, idx1_hbm, g1_hbm, g2_hbm,
                 idx1_v, idx2_v, rows1_v, rows2_v,
                 sem_g1, sem_g2, sem_w1, sem_w2):
    wid = lax.axis_index("s") * 2 + lax.axis_index("c")

    def body(i, carry):
        chunk = wid + NW * i

        @pl.when(chunk < NCHUNK)
        def _():
            base = chunk * GC
            pltpu.sync_copy(seg_hbm.at[pl.ds(base, GC)], idx1_v)
            g1 = pltpu.async_copy(n2_hbm.at[idx1_v], rows1_v, sem_g1)
            pltpu.sync_copy(idx1_hbm.at[pl.ds(base, GC)], idx2_v)
            g2 = pltpu.async_copy(nout_hbm.at[idx2_v], rows2_v, sem_g2)
            g1.wait()
            w1 = pltpu.async_copy(rows1_v, g1_hbm.at[pl.ds(base, GC)], sem_w1)
            g2.wait()
            w2 = pltpu.async_copy(rows2_v, g2_hbm.at[pl.ds(base, GC)], sem_w2)
            w1.wait()
            w2.wait()

        return carry

    lax.fori_loop(0, (NCHUNK + NW - 1) // NW, body, 0)


def _sc_gather(n2, n_out, seg, idx1):
    mesh = plsc.VectorSubcoreMesh(core_axis_name="c", subcore_axis_name="s")
    return pl.kernel(
        _gather_body,
        out_type=(
            jax.ShapeDtypeStruct((NEDGES, 128), F32),
            jax.ShapeDtypeStruct((NEDGES, 128), F32),
        ),
        mesh=mesh,
        scratch_types=[
            pltpu.VMEM((GC,), jnp.int32),
            pltpu.VMEM((GC,), jnp.int32),
            pltpu.VMEM((GC, 128), F32),
            pltpu.VMEM((GC, 128), F32),
            pltpu.SemaphoreType.DMA,
            pltpu.SemaphoreType.DMA,
            pltpu.SemaphoreType.DMA,
            pltpu.SemaphoreType.DMA,
        ],
    )(n2, n_out, seg, idx1)


EBLK = 4000
_INV_SQRT2 = 0.7071067811865476


def _stage_c_body(edges, eenc, g1, g2, we, be, p_mat, weo, beo, wc, c2, r_mat,
                  ne_o, scaled_o, w_o):
    e = edges[...] + eenc[...]
    ep = jnp.dot(e, we[...], preferred_element_type=F32) + be[...]
    x = _unpack_bf16_pairs(g1[...]) + ep
    h = x * 0.5 * (1.0 + lax.erf(x * _INV_SQRT2))
    logits = jnp.dot(h, p_mat[...], preferred_element_type=F32)
    w = jnp.exp(logits)
    ne_o[...] = jnp.dot(h, weo[...], preferred_element_type=F32) + beo[...]
    vpre = (_unpack_bf16_pairs(g2[...])
            + jnp.dot(h, wc[...], preferred_element_type=F32) + c2[...])
    wrep = jnp.dot(w, r_mat[...], preferred_element_type=F32)
    scaled = vpre * wrep
    scaled_o[0, :, :] = scaled[:, :128]
    scaled_o[1, :, :] = scaled[:, 128:]
    w_o[...] = w


def _stage_c(edges, eenc, g1, g2, we, be, p_mat, weo, beo, wc, c2, r_mat):
    grid = (NEDGES // EBLK,)
    eb = lambda i: (i, 0)
    full = lambda i: (0, 0)
    return pl.pallas_call(
        _stage_c_body,
        grid=grid,
        in_specs=[
            pl.BlockSpec((EBLK, HID), eb),
            pl.BlockSpec((EBLK, HID), eb),
            pl.BlockSpec((EBLK, 128), eb),
            pl.BlockSpec((EBLK, 128), eb),
            pl.BlockSpec((HID, HID), full),
            pl.BlockSpec((1, HID), full),
            pl.BlockSpec((HID, 16), full),
            pl.BlockSpec((HID, HID), full),
            pl.BlockSpec((1, HID), full),
            pl.BlockSpec((HID, HID), full),
            pl.BlockSpec((1, HID), full),
            pl.BlockSpec((16, HID), full),
        ],
        out_specs=[
            pl.BlockSpec((EBLK, HID), eb),
            pl.BlockSpec((2, EBLK, 128), lambda i: (0, i, 0)),
            pl.BlockSpec((EBLK, 16), eb),
        ],
        out_shape=(
            jax.ShapeDtypeStruct((NEDGES, HID), F32),
            jax.ShapeDtypeStruct((2, NEDGES, 128), F32),
            jax.ShapeDtypeStruct((NEDGES, 16), F32),
        ),
    )(edges, eenc, g1, g2, we, be, p_mat, weo, beo, wc, c2, r_mat)


SGC = 64
SNCHUNK = NEDGES // SGC
ENODES = NNODES // 8
SZCHUNK = NNODES // SGC
SZTAIL = NNODES - SZCHUNK * SGC
SNHJ = SGC // 16


def _scatter_body(scaled_hbm, wflat_hbm, seg_hbm, z128_hbm, zden_hbm,
                  num_o, den_o,
                  idx0_v, idx1_v, rows0_v, rows1_v, wf0_v, wf1_v, den_acc,
                  num_sp,
                  sem_i0, sem_i1, sem_w0, sem_w1, sem_r0, sem_r1,
                  sem_s0, sem_s1):
    c = lax.axis_index("c")
    s = lax.axis_index("s")
    slots = ((idx0_v, rows0_v, wf0_v, sem_i0, sem_w0, sem_r0, sem_s0),
             (idx1_v, rows1_v, wf1_v, sem_i1, sem_w1, sem_r1, sem_s1))

    pltpu.sync_copy(z128_hbm.at[pl.ds(0, SGC)], rows0_v)

    def zbody(i, carry):
        chunk = s + 16 * i

        @pl.when(chunk < SZCHUNK)
        def _():
            pltpu.sync_copy(rows0_v, num_sp.at[pl.ds(chunk * SGC, SGC)])

        return carry

    lax.fori_loop(0, (SZCHUNK + 15) // 16, zbody, 0)

    @pl.when(s == 15)
    def _():
        pltpu.sync_copy(rows0_v.at[pl.ds(0, SZTAIL)],
                        num_sp.at[pl.ds(SZCHUNK * SGC, SZTAIL)])

    plsc.subcore_barrier()

    lane = lax.iota(jnp.int32, 16)

    def chunk_of(k):
        return s + 16 * k

    def start_loads(k, slot, with_num):
        idx_v, rows_v, wf_v, sem_i, sem_w, sem_r, _ = slots[slot]

        @pl.when(chunk_of(k) < SNCHUNK)
        def _():
            base = chunk_of(k) * SGC
            pltpu.async_copy(seg_hbm.at[pl.ds(base, SGC)], idx_v, sem_i)
            pltpu.async_copy(wflat_hbm.at[pl.ds(base * 16, SGC * 16)],
                             wf_v, sem_w)
            if with_num:
                pltpu.async_copy(scaled_hbm.at[c, pl.ds(base, SGC)],
                                 rows_v, sem_r)

    def run_pass(q, with_num):
        region_base = (4 * c + q) * ENODES
        pltpu.sync_copy(zden_hbm, den_acc)
        for slot in (0, 1):
            start_loads(slot, slot, with_num)

        def half_step(k, slot):
            idx_v, rows_v, wf_v, sem_i, sem_w, sem_r, sem_s = slots[slot]

            @pl.when(chunk_of(k) < SNCHUNK)
            def _():
                base = chunk_of(k) * SGC
                pltpu.make_async_copy(seg_hbm.at[pl.ds(base, SGC)], idx_v,
                                      sem_i).wait()
                pltpu.make_async_copy(
                    wflat_hbm.at[pl.ds(base * 16, SGC * 16)], wf_v,
                    sem_w).wait()
                if with_num:
                    pltpu.make_async_copy(scaled_hbm.at[c, pl.ds(base, SGC)],
                                          rows_v, sem_r).wait()
                    pltpu.async_copy(rows_v, num_sp.at[idx_v], sem_s,
                                     add=True)

                def dbody(j, carry2):
                    ids = idx_v[pl.ds(j * 16, 16)]
                    t = ids - region_base
                    ok = (t >= 0) & (t < ENODES)
                    tc16 = jnp.where(ok, t, 0) * 16
                    rows16 = (j * 16 + lane) * 16
                    for h in range(NHEADS):
                        vals = plsc.load_gather(wf_v, [rows16 + h])
                        plsc.addupdate_scatter(den_acc, [tc16 + h], vals,
                                               mask=ok)
                    return carry2

                lax.fori_loop(0, SNHJ, dbody, 0)
                if with_num:
                    pltpu.make_async_copy(rows_v, num_sp.at[idx_v],
                                          sem_s).wait()
                start_loads(k + 2, slot, with_num)

        def body(m, carry):
            half_step(2 * m, 0)
            half_step(2 * m + 1, 1)
            return carry

        nk = (SNCHUNK + 15) // 16
        lax.fori_loop(0, (nk + 1) // 2, body, 0)
        pltpu.sync_copy(den_acc, den_o.at[c, q, s])

    run_pass(0, True)
    for q in (1, 2, 3):
        run_pass(q, False)

    plsc.subcore_barrier()

    def fbody(i, carry):
        chunk = s + 16 * i

        @pl.when(chunk < SZCHUNK)
        def _():
            base = chunk * SGC
            pltpu.sync_copy(num_sp.at[pl.ds(base, SGC)], rows0_v)
            pltpu.sync_copy(rows0_v, num_o.at[c, pl.ds(base, SGC)])

        return carry

    lax.fori_loop(0, (SZCHUNK + 15) // 16, fbody, 0)

    @pl.when(s == 15)
    def _():
        pltpu.sync_copy(num_sp.at[pl.ds(SZCHUNK * SGC, SZTAIL)],
                        rows0_v.at[pl.ds(0, SZTAIL)])
        pltpu.sync_copy(rows0_v.at[pl.ds(0, SZTAIL)],
                        num_o.at[c, pl.ds(SZCHUNK * SGC, SZTAIL)])


def _sc_scatter(scaled2, wflat, seg, z128, zden):
    mesh = plsc.VectorSubcoreMesh(core_axis_name="c", subcore_axis_name="s")
    return pl.kernel(
        _scatter_body,
        out_type=(
            jax.ShapeDtypeStruct((2, NNODES, 128), F32),
            jax.ShapeDtypeStruct((2, 4, 16, ENODES * 16), F32),
        ),
        mesh=mesh,
        compiler_params=pltpu.CompilerParams(needs_layout_passes=False),
        scratch_types=[
            pltpu.VMEM((SGC,), jnp.int32),
            pltpu.VMEM((SGC,), jnp.int32),
            pltpu.VMEM((SGC, 128), F32),
            pltpu.VMEM((SGC, 128), F32),
            pltpu.VMEM((SGC * 16,), F32),
            pltpu.VMEM((SGC * 16,), F32),
            pltpu.VMEM((ENODES * 16,), F32),
            pltpu.VMEM_SHARED((NNODES, 128), F32),
            pltpu.SemaphoreType.DMA,
            pltpu.SemaphoreType.DMA,
            pltpu.SemaphoreType.DMA,
            pltpu.SemaphoreType.DMA,
            pltpu.SemaphoreType.DMA,
            pltpu.SemaphoreType.DMA,
            pltpu.SemaphoreType.DMA,
            pltpu.SemaphoreType.DMA,
        ],
    )(scaled2, wflat, seg, z128, zden)


def _stage_d2_body(parts, den_o):
    den_o[0] = jnp.sum(parts[0, 0], axis=0)


def _stage_d2(den_parts):
    return pl.pallas_call(
        _stage_d2_body,
        grid=(8,),
        in_specs=[pl.BlockSpec((1, 1, 16, ENODES, 16),
                               lambda i: (i // 4, i % 4, 0, 0, 0))],
        out_specs=pl.BlockSpec((1, ENODES, 16), lambda i: (i, 0, 0)),
        out_shape=jax.ShapeDtypeStruct((8, ENODES, 16), F32),
    )(den_parts)


NBLK = 2000


def _stage_e_body(num0, num1, den, r0, r1, out_o):
    d = den[...]
    d0 = jnp.dot(d, r0[...], preferred_element_type=F32)
    d1 = jnp.dot(d, r1[...], preferred_element_type=F32)
    rec0 = jnp.where(d0 > 0, 1.0 / d0, 0.0)
    rec1 = jnp.where(d1 > 0, 1.0 / d1, 0.0)
    out_o[:, 0:128] = num0[0] * rec0
    out_o[:, 128:256] = num1[0] * rec1


def _stage_e(num2, den, r0, r1):
    grid = (NNODES // NBLK,)
    return pl.pallas_call(
        _stage_e_body,
        grid=grid,
        in_specs=[
            pl.BlockSpec((1, NBLK, 128), lambda i: (0, i, 0)),
            pl.BlockSpec((1, NBLK, 128), lambda i: (1, i, 0)),
            pl.BlockSpec((NBLK, 16), lambda i: (i, 0)),
            pl.BlockSpec((16, 128), lambda i: (0, 0)),
            pl.BlockSpec((16, 128), lambda i: (0, 0)),
        ],
        out_specs=pl.BlockSpec((NBLK, HID), lambda i: (i, 0)),
        out_shape=jax.ShapeDtypeStruct((NNODES, HID), F32),
    )(num2, num2, den, r0, r1)


_R_MAT_NP = np.zeros((16, HID), np.float32)
for _h in range(NHEADS):
    _R_MAT_NP[_h, _h * HD:(_h + 1) * HD] = 1.0
_R0_NP = np.zeros((16, 128), np.float32)
for _h in range(4):
    _R0_NP[_h, _h * HD:(_h + 1) * HD] = 1.0
_R1_NP = np.zeros((16, 128), np.float32)
for _h in range(4):
    _R1_NP[_h + 4, _h * HD:(_h + 1) * HD] = 1.0


def kernel(nodes, edges, edge_index, node_encodings, edge_encodings,
           W_first, b_first, W_second, b_second, W_edge, b_edge,
           attn_proj, W_edge_out, b_edge_out, W_node_out, b_node_out):
    seg = edge_index[0]
    idx1 = edge_index[1]

    b12 = (b_first + b_second).reshape(1, HID)
    be = b_edge.reshape(1, HID)
    beo = b_edge_out.reshape(1, HID)
    bno = b_node_out.reshape(1, HID)

    a = attn_proj[0]
    p_mat = (a[:, :, None] * jnp.eye(NHEADS, dtype=F32)[:, None, :])
    p_mat = p_mat.reshape(HID, NHEADS)
    p_mat = jnp.concatenate([p_mat, jnp.zeros((HID, 8), F32)], axis=1)

    n2, n_out, wc, c2 = _stage_a(nodes, node_encodings, W_first, W_second,
                                 b12, W_node_out, W_edge_out, beo, bno)
    g1, g2 = _sc_gather(n2, n_out, seg, idx1)
    new_edges, scaled2, w16 = _stage_c(edges, edge_encodings, g1, g2,
                                       W_edge, be, p_mat, W_edge_out, beo,
                                       wc, c2, jnp.asarray(_R_MAT_NP))
    z128 = jnp.zeros((NNODES, 128), F32)
    zden = jnp.zeros((ENODES * 16,), F32)
    num2, den_parts = _sc_scatter(scaled2, w16.reshape(-1), seg, z128, zden)
    den = _stage_d2(
        den_parts.reshape(2, 4, 16, ENODES, 16)).reshape(NNODES, 16)
    new_nodes = _stage_e(num2, den, jnp.asarray(_R0_NP), jnp.asarray(_R1_NP))
    return new_nodes, new_edges


# speedup vs baseline: 1.0561x; 1.0365x over previous
import numpy as np
import jax
import jax.numpy as jnp
from jax import lax
from jax.experimental import pallas as pl
from jax.experimental.pallas import tpu as pltpu
from jax.experimental.pallas import tpu_sc as plsc

HID = 256
NHEADS = 8
HD = 32
NNODES = 10000
NEDGES = 160000

F32 = jnp.float32


def _pack_bf16_pairs(x):
    lo = lax.bitcast_convert_type(x[:, :128], jnp.uint32)
    hi = lax.bitcast_convert_type(x[:, 128:], jnp.uint32)
    lo16 = (lo + 0x8000) >> 16
    hi16 = (hi + 0x8000) & jnp.uint32(0xFFFF0000)
    return lax.bitcast_convert_type(lo16 | hi16, F32)


def _unpack_bf16_pairs(p):
    u = lax.bitcast_convert_type(p, jnp.uint32)
    lo = lax.bitcast_convert_type(u << 16, F32)
    hi = lax.bitcast_convert_type(u & jnp.uint32(0xFFFF0000), F32)
    return jnp.concatenate([lo, hi], axis=1)


def _stage_a_body(nodes, nenc, wf, ws, b12, wno, weo, beo, bno,
                  n2_o, nout_o, wc_o, c2_o):
    n = nodes[...] + nenc[...]
    w12 = wf[...] + ws[...]
    n2 = jnp.dot(n, w12, preferred_element_type=F32) + b12[...]
    n2_o[...] = _pack_bf16_pairs(n2)
    n_out = jnp.dot(n, wno[...], preferred_element_type=F32)
    nout_o[...] = _pack_bf16_pairs(n_out)
    wc_o[...] = jnp.dot(weo[...], wno[...], preferred_element_type=F32)
    c2_o[...] = jnp.dot(beo[...], wno[...], preferred_element_type=F32) + bno[...]


def _stage_a(nodes, nenc, wf, ws, b12, wno, weo, beo, bno):
    return pl.pallas_call(
        _stage_a_body,
        out_shape=(
            jax.ShapeDtypeStruct((NNODES, 128), F32),
            jax.ShapeDtypeStruct((NNODES, 128), F32),
            jax.ShapeDtypeStruct((HID, HID), F32),
            jax.ShapeDtypeStruct((1, HID), F32),
        ),
    )(nodes, nenc, wf, ws, b12, wno, weo, beo, bno)


GC = 256
NCHUNK = NEDGES // GC
NW = 32


def _gather_body(n2_hbm, nout_hbm, seg_hbm, idx1_hbm, g1_hbm, g2_hbm,
                 idx1_v, idx2_v, rows1_v, rows2_v,
                 sem_g1, sem_g2, sem_w1, sem_w2):
    wid = lax.axis_index("s") * 2 + lax.axis_index("c")

    def body(i, carry):
        chunk = wid + NW * i

        @pl.when(chunk < NCHUNK)
        def _():
            base = chunk * GC
            pltpu.sync_copy(seg_hbm.at[pl.ds(base, GC)], idx1_v)
            g1 = pltpu.async_copy(n2_hbm.at[idx1_v], rows1_v, sem_g1)
            pltpu.sync_copy(idx1_hbm.at[pl.ds(base, GC)], idx2_v)
            g2 = pltpu.async_copy(nout_hbm.at[idx2_v], rows2_v, sem_g2)
            g1.wait()
            w1 = pltpu.async_copy(rows1_v, g1_hbm.at[pl.ds(base, GC)], sem_w1)
            g2.wait()
            w2 = pltpu.async_copy(rows2_v, g2_hbm.at[pl.ds(base, GC)], sem_w2)
            w1.wait()
            w2.wait()

        return carry

    lax.fori_loop(0, (NCHUNK + NW - 1) // NW, body, 0)


def _sc_gather(n2, n_out, seg, idx1):
    mesh = plsc.VectorSubcoreMesh(core_axis_name="c", subcore_axis_name="s")
    return pl.kernel(
        _gather_body,
        out_type=(
            jax.ShapeDtypeStruct((NEDGES, 128), F32),
            jax.ShapeDtypeStruct((NEDGES, 128), F32),
        ),
        mesh=mesh,
        scratch_types=[
            pltpu.VMEM((GC,), jnp.int32),
            pltpu.VMEM((GC,), jnp.int32),
            pltpu.VMEM((GC, 128), F32),
            pltpu.VMEM((GC, 128), F32),
            pltpu.SemaphoreType.DMA,
            pltpu.SemaphoreType.DMA,
            pltpu.SemaphoreType.DMA,
            pltpu.SemaphoreType.DMA,
        ],
    )(n2, n_out, seg, idx1)


EBLK = 4000
_INV_SQRT2 = 0.7071067811865476


def _stage_c_body(edges, eenc, g1, g2, we, be, p_mat, weo, beo, wc, c2, r_mat,
                  ne_o, scaled_o, w_o):
    e = edges[...] + eenc[...]
    ep = jnp.dot(e, we[...], preferred_element_type=F32) + be[...]
    x = _unpack_bf16_pairs(g1[...]) + ep
    h = x * 0.5 * (1.0 + lax.erf(x * _INV_SQRT2))
    logits = jnp.dot(h, p_mat[...], preferred_element_type=F32)
    w = jnp.exp(logits)
    ne_o[...] = jnp.dot(h, weo[...], preferred_element_type=F32) + beo[...]
    vpre = (_unpack_bf16_pairs(g2[...])
            + jnp.dot(h, wc[...], preferred_element_type=F32) + c2[...])
    wrep = jnp.dot(w, r_mat[...], preferred_element_type=F32)
    scaled = vpre * wrep
    scaled_o[0, :, :] = scaled[:, :128]
    scaled_o[1, :, :] = scaled[:, 128:]
    w_o[...] = w


def _stage_c(edges, eenc, g1, g2, we, be, p_mat, weo, beo, wc, c2, r_mat):
    grid = (NEDGES // EBLK,)
    eb = lambda i: (i, 0)
    full = lambda i: (0, 0)
    return pl.pallas_call(
        _stage_c_body,
        grid=grid,
        in_specs=[
            pl.BlockSpec((EBLK, HID), eb),
            pl.BlockSpec((EBLK, HID), eb),
            pl.BlockSpec((EBLK, 128), eb),
            pl.BlockSpec((EBLK, 128), eb),
            pl.BlockSpec((HID, HID), full),
            pl.BlockSpec((1, HID), full),
            pl.BlockSpec((HID, 16), full),
            pl.BlockSpec((HID, HID), full),
            pl.BlockSpec((1, HID), full),
            pl.BlockSpec((HID, HID), full),
            pl.BlockSpec((1, HID), full),
            pl.BlockSpec((16, HID), full),
        ],
        out_specs=[
            pl.BlockSpec((EBLK, HID), eb),
            pl.BlockSpec((2, EBLK, 128), lambda i: (0, i, 0)),
            pl.BlockSpec((EBLK, 16), eb),
        ],
        out_shape=(
            jax.ShapeDtypeStruct((NEDGES, HID), F32),
            jax.ShapeDtypeStruct((2, NEDGES, 128), F32),
            jax.ShapeDtypeStruct((NEDGES, 16), F32),
        ),
    )(edges, eenc, g1, g2, we, be, p_mat, weo, beo, wc, c2, r_mat)


SGC = 80
SNCHUNK = NEDGES // SGC
ENODES = NNODES // 8
SZCHUNK = NNODES // SGC
SZTAIL = NNODES - SZCHUNK * SGC
SNHJ = SGC // 16


def _scatter_body(scaled_hbm, wflat_hbm, seg_hbm, z128_hbm, zden_hbm,
                  num_o, den_o,
                  idx0_v, idx1_v, rows0_v, rows1_v, wf0_v, wf1_v, den_acc,
                  num_sp,
                  sem_i0, sem_i1, sem_w0, sem_w1, sem_r0, sem_r1,
                  sem_s0, sem_s1):
    c = lax.axis_index("c")
    s = lax.axis_index("s")
    slots = ((idx0_v, rows0_v, wf0_v, sem_i0, sem_w0, sem_r0, sem_s0),
             (idx1_v, rows1_v, wf1_v, sem_i1, sem_w1, sem_r1, sem_s1))

    pltpu.sync_copy(z128_hbm.at[pl.ds(0, SGC)], rows0_v)

    def zbody(i, carry):
        chunk = s + 16 * i

        @pl.when(chunk < SZCHUNK)
        def _():
            pltpu.sync_copy(rows0_v, num_sp.at[pl.ds(chunk * SGC, SGC)])

        return carry

    lax.fori_loop(0, (SZCHUNK + 15) // 16, zbody, 0)

    if SZTAIL:
        @pl.when(s == 15)
        def _():
            pltpu.sync_copy(rows0_v.at[pl.ds(0, SZTAIL)],
                            num_sp.at[pl.ds(SZCHUNK * SGC, SZTAIL)])

    plsc.subcore_barrier()

    lane = lax.iota(jnp.int32, 16)

    def chunk_of(k):
        return s + 16 * k

    def start_loads(k, slot, with_num):
        idx_v, rows_v, wf_v, sem_i, sem_w, sem_r, _ = slots[slot]

        @pl.when(chunk_of(k) < SNCHUNK)
        def _():
            base = chunk_of(k) * SGC
            pltpu.async_copy(seg_hbm.at[pl.ds(base, SGC)], idx_v, sem_i)
            pltpu.async_copy(wflat_hbm.at[pl.ds(base * 16, SGC * 16)],
                             wf_v, sem_w)
            if with_num:
                pltpu.async_copy(scaled_hbm.at[c, pl.ds(base, SGC)],
                                 rows_v, sem_r)

    def run_pass(q, with_num):
        region_base = (4 * c + q) * ENODES
        pltpu.sync_copy(zden_hbm, den_acc)
        for slot in (0, 1):
            start_loads(slot, slot, with_num)

        def half_step(k, slot):
            idx_v, rows_v, wf_v, sem_i, sem_w, sem_r, sem_s = slots[slot]

            @pl.when(chunk_of(k) < SNCHUNK)
            def _():
                base = chunk_of(k) * SGC
                pltpu.make_async_copy(seg_hbm.at[pl.ds(base, SGC)], idx_v,
                                      sem_i).wait()
                pltpu.make_async_copy(
                    wflat_hbm.at[pl.ds(base * 16, SGC * 16)], wf_v,
                    sem_w).wait()
                if with_num:
                    pltpu.make_async_copy(scaled_hbm.at[c, pl.ds(base, SGC)],
                                          rows_v, sem_r).wait()
                    pltpu.async_copy(rows_v, num_sp.at[idx_v], sem_s,
                                     add=True)

                def dbody(j, carry2):
                    ids = idx_v[pl.ds(j * 16, 16)]
                    t = ids - region_base
                    ok = (t >= 0) & (t < ENODES)
                    tc16 = jnp.where(ok, t, 0) * 16
                    rows16 = (j * 16 + lane) * 16
                    for h in range(NHEADS):
                        vals = plsc.load_gather(wf_v, [rows16 + h])
                        plsc.addupdate_scatter(den_acc, [tc16 + h], vals,
                                               mask=ok)
                    return carry2

                lax.fori_loop(0, SNHJ, dbody, 0)
                if with_num:
                    pltpu.make_async_copy(rows_v, num_sp.at[idx_v],
                                          sem_s).wait()
                start_loads(k + 2, slot, with_num)

        def body(m, carry):
            half_step(2 * m, 0)
            half_step(2 * m + 1, 1)
            return carry

        nk = (SNCHUNK + 15) // 16
        lax.fori_loop(0, (nk + 1) // 2, body, 0)
        pltpu.sync_copy(den_acc, den_o.at[c, q, s])

    run_pass(0, True)
    for q in (1, 2, 3):
        run_pass(q, False)

    plsc.subcore_barrier()

    def fbody(i, carry):
        chunk = s + 16 * i

        @pl.when(chunk < SZCHUNK)
        def _():
            base = chunk * SGC
            pltpu.sync_copy(num_sp.at[pl.ds(base, SGC)], rows0_v)
            pltpu.sync_copy(rows0_v, num_o.at[c, pl.ds(base, SGC)])

        return carry

    lax.fori_loop(0, (SZCHUNK + 15) // 16, fbody, 0)

    if SZTAIL:
        @pl.when(s == 15)
        def _():
            pltpu.sync_copy(num_sp.at[pl.ds(SZCHUNK * SGC, SZTAIL)],
                            rows0_v.at[pl.ds(0, SZTAIL)])
            pltpu.sync_copy(rows0_v.at[pl.ds(0, SZTAIL)],
                            num_o.at[c, pl.ds(SZCHUNK * SGC, SZTAIL)])


def _sc_scatter(scaled2, wflat, seg, z128, zden):
    mesh = plsc.VectorSubcoreMesh(core_axis_name="c", subcore_axis_name="s")
    return pl.kernel(
        _scatter_body,
        out_type=(
            jax.ShapeDtypeStruct((2, NNODES, 128), F32),
            jax.ShapeDtypeStruct((2, 4, 16, ENODES * 16), F32),
        ),
        mesh=mesh,
        compiler_params=pltpu.CompilerParams(needs_layout_passes=False),
        scratch_types=[
            pltpu.VMEM((SGC,), jnp.int32),
            pltpu.VMEM((SGC,), jnp.int32),
            pltpu.VMEM((SGC, 128), F32),
            pltpu.VMEM((SGC, 128), F32),
            pltpu.VMEM((SGC * 16,), F32),
            pltpu.VMEM((SGC * 16,), F32),
            pltpu.VMEM((ENODES * 16,), F32),
            pltpu.VMEM_SHARED((NNODES, 128), F32),
            pltpu.SemaphoreType.DMA,
            pltpu.SemaphoreType.DMA,
            pltpu.SemaphoreType.DMA,
            pltpu.SemaphoreType.DMA,
            pltpu.SemaphoreType.DMA,
            pltpu.SemaphoreType.DMA,
            pltpu.SemaphoreType.DMA,
            pltpu.SemaphoreType.DMA,
        ],
    )(scaled2, wflat, seg, z128, zden)


def _stage_d2_body(parts, den_o):
    den_o[0] = jnp.sum(parts[0, 0], axis=0)


def _stage_d2(den_parts):
    return pl.pallas_call(
        _stage_d2_body,
        grid=(8,),
        in_specs=[pl.BlockSpec((1, 1, 16, ENODES, 16),
                               lambda i: (i // 4, i % 4, 0, 0, 0))],
        out_specs=pl.BlockSpec((1, ENODES, 16), lambda i: (i, 0, 0)),
        out_shape=jax.ShapeDtypeStruct((8, ENODES, 16), F32),
    )(den_parts)


NBLK = 2000


def _stage_e_body(num0, num1, den, r0, r1, out_o):
    d = den[...]
    d0 = jnp.dot(d, r0[...], preferred_element_type=F32)
    d1 = jnp.dot(d, r1[...], preferred_element_type=F32)
    rec0 = jnp.where(d0 > 0, 1.0 / d0, 0.0)
    rec1 = jnp.where(d1 > 0, 1.0 / d1, 0.0)
    out_o[:, 0:128] = num0[0] * rec0
    out_o[:, 128:256] = num1[0] * rec1


def _stage_e(num2, den, r0, r1):
    grid = (NNODES // NBLK,)
    return pl.pallas_call(
        _stage_e_body,
        grid=grid,
        in_specs=[
            pl.BlockSpec((1, NBLK, 128), lambda i: (0, i, 0)),
            pl.BlockSpec((1, NBLK, 128), lambda i: (1, i, 0)),
            pl.BlockSpec((NBLK, 16), lambda i: (i, 0)),
            pl.BlockSpec((16, 128), lambda i: (0, 0)),
            pl.BlockSpec((16, 128), lambda i: (0, 0)),
        ],
        out_specs=pl.BlockSpec((NBLK, HID), lambda i: (i, 0)),
        out_shape=jax.ShapeDtypeStruct((NNODES, HID), F32),
    )(num2, num2, den, r0, r1)


_R_MAT_NP = np.zeros((16, HID), np.float32)
for _h in range(NHEADS):
    _R_MAT_NP[_h, _h * HD:(_h + 1) * HD] = 1.0
_R0_NP = np.zeros((16, 128), np.float32)
for _h in range(4):
    _R0_NP[_h, _h * HD:(_h + 1) * HD] = 1.0
_R1_NP = np.zeros((16, 128), np.float32)
for _h in range(4):
    _R1_NP[_h + 4, _h * HD:(_h + 1) * HD] = 1.0


def kernel(nodes, edges, edge_index, node_encodings, edge_encodings,
           W_first, b_first, W_second, b_second, W_edge, b_edge,
           attn_proj, W_edge_out, b_edge_out, W_node_out, b_node_out):
    seg = edge_index[0]
    idx1 = edge_index[1]

    b12 = (b_first + b_second).reshape(1, HID)
    be = b_edge.reshape(1, HID)
    beo = b_edge_out.reshape(1, HID)
    bno = b_node_out.reshape(1, HID)

    a = attn_proj[0]
    p_mat = (a[:, :, None] * jnp.eye(NHEADS, dtype=F32)[:, None, :])
    p_mat = p_mat.reshape(HID, NHEADS)
    p_mat = jnp.concatenate([p_mat, jnp.zeros((HID, 8), F32)], axis=1)

    n2, n_out, wc, c2 = _stage_a(nodes, node_encodings, W_first, W_second,
                                 b12, W_node_out, W_edge_out, beo, bno)
    g1, g2 = _sc_gather(n2, n_out, seg, idx1)
    new_edges, scaled2, w16 = _stage_c(edges, edge_encodings, g1, g2,
                                       W_edge, be, p_mat, W_edge_out, beo,
                                       wc, c2, jnp.asarray(_R_MAT_NP))
    z128 = jnp.zeros((NNODES, 128), F32)
    zden = jnp.zeros((ENODES * 16,), F32)
    num2, den_parts = _sc_scatter(scaled2, w16.reshape(-1), seg, z128, zden)
    den = _stage_d2(
        den_parts.reshape(2, 4, 16, ENODES, 16)).reshape(NNODES, 16)
    new_nodes = _stage_e(num2, den, jnp.asarray(_R0_NP), jnp.asarray(_R1_NP))
    return new_nodes, new_edges
